# initial kernel scaffold (unmeasured)
import jax
import jax.numpy as jnp
from jax import lax
from jax.experimental import pallas as pl
from jax.experimental.pallas import tpu as pltpu

N_DEV = 4


def kernel(Q, K, V):
    b, s_per, h, d = Q.shape
    g = b * h
    scale = d ** -0.5

    def to_groups(x):
        return x.transpose(0, 2, 1, 3).reshape(g, s_per, d)

    Qg = to_groups(Q)
    KVg = jnp.concatenate([to_groups(K), to_groups(V)], axis=0)

    def body(q_ref, kv_ref, out_ref, kv_all, send_sems, recv_sems):
        my_pos = lax.axis_index("i")
        left = lax.rem(my_pos - 1 + N_DEV, N_DEV)
        right = lax.rem(my_pos + 1, N_DEV)

        barrier_sem = pltpu.get_barrier_semaphore()
        for nbr in (left, right):
            pl.semaphore_signal(
                barrier_sem, inc=1,
                device_id=(nbr,), device_id_type=pl.DeviceIdType.MESH,
            )
        pl.semaphore_wait(barrier_sem, 2)

        kv_all[my_pos] = kv_ref[...]

        for hop in range(N_DEV - 1):
            o_send = lax.rem(my_pos - hop + N_DEV, N_DEV)
            rdma = pltpu.make_async_remote_copy(
                src_ref=kv_all.at[o_send],
                dst_ref=kv_all.at[o_send],
                send_sem=send_sems.at[hop],
                recv_sem=recv_sems.at[hop],
                device_id=(right,),
                device_id_type=pl.DeviceIdType.MESH,
            )
            rdma.start()
            rdma.wait()

        q = q_ref[...]
        dn_qk = (((2,), (2,)), ((0,), (0,)))
        s_parts = [
            lax.dot_general(
                q, kv_all[o, :g], dn_qk, preferred_element_type=jnp.float32
            )
            for o in range(N_DEV)
        ]
        s_full = jnp.concatenate(s_parts, axis=2) * scale
        m = jnp.max(s_full, axis=2, keepdims=True)
        p = jnp.exp(s_full - m)
        l = jnp.sum(p, axis=2, keepdims=True)
        dn_pv = (((2,), (1,)), ((0,), (0,)))
        acc = None
        for o in range(N_DEV):
            pv = lax.dot_general(
                p[:, :, o * s_per:(o + 1) * s_per],
                kv_all[o, g:],
                dn_pv,
                preferred_element_type=jnp.float32,
            )
            acc = pv if acc is None else acc + pv
        out_ref[...] = acc / l

    out = pl.pallas_call(
        body,
        out_shape=jax.ShapeDtypeStruct((g, s_per, d), jnp.float32),
        in_specs=[
            pl.BlockSpec(memory_space=pltpu.VMEM),
            pl.BlockSpec(memory_space=pltpu.VMEM),
        ],
        out_specs=pl.BlockSpec(memory_space=pltpu.VMEM),
        scratch_shapes=[
            pltpu.VMEM((N_DEV, 2 * g, s_per, d), jnp.float32),
            pltpu.SemaphoreType.DMA((N_DEV - 1,)),
            pltpu.SemaphoreType.DMA((N_DEV - 1,)),
        ],
        compiler_params=pltpu.CompilerParams(collective_id=0),
    )(Qg, KVg)

    return out.reshape(b, h, s_per, d).transpose(0, 2, 1, 3)


# baseline (device time: 158586 ns/iter reference)
import jax
import jax.numpy as jnp
from jax import lax
from jax.experimental import pallas as pl
from jax.experimental.pallas import tpu as pltpu

N_DEV = 4


def kernel(Q, K, V):
    b, s_per, h, d = Q.shape
    g = b * h
    scale = d ** -0.5

    def to_groups(x):
        return x.transpose(0, 2, 1, 3).reshape(g, s_per, d)

    Qg = to_groups(Q)
    KVg = jnp.concatenate([to_groups(K), to_groups(V)], axis=0)

    def body(q_ref, kv_ref, out_ref, kv_all, send_sems, recv_sems):
        my_pos = lax.axis_index("i")
        left = lax.rem(my_pos - 1 + N_DEV, N_DEV)
        right = lax.rem(my_pos + 1, N_DEV)

        barrier_sem = pltpu.get_barrier_semaphore()
        for nbr in (left, right):
            pl.semaphore_signal(
                barrier_sem, inc=1,
                device_id=(nbr,), device_id_type=pl.DeviceIdType.MESH,
            )
        pl.semaphore_wait(barrier_sem, 2)

        kv_all[my_pos] = kv_ref[...]

        for hop in range(N_DEV - 1):
            o_send = lax.rem(my_pos - hop + N_DEV, N_DEV)
            rdma = pltpu.make_async_remote_copy(
                src_ref=kv_all.at[o_send],
                dst_ref=kv_all.at[o_send],
                send_sem=send_sems.at[hop],
                recv_sem=recv_sems.at[hop],
                device_id=(right,),
                device_id_type=pl.DeviceIdType.MESH,
            )
            rdma.start()
            rdma.wait()

        q = q_ref[...]
        dn_qk = (((2,), (2,)), ((0,), (0,)))
        dn_pv = (((2,), (1,)), ((0,), (0,)))
        m_run = jnp.full((g, s_per, 1), -jnp.inf, jnp.float32)
        l_run = jnp.zeros((g, s_per, 1), jnp.float32)
        acc = jnp.zeros((g, s_per, d), jnp.float32)
        for o in range(N_DEV):
            s_o = lax.dot_general(
                q, kv_all[o, :g], dn_qk, preferred_element_type=jnp.float32
            ) * scale
            m_new = jnp.maximum(m_run, jnp.max(s_o, axis=2, keepdims=True))
            alpha = jnp.exp(m_run - m_new)
            p_o = jnp.exp(s_o - m_new)
            l_run = l_run * alpha + jnp.sum(p_o, axis=2, keepdims=True)
            acc = acc * alpha + lax.dot_general(
                p_o, kv_all[o, g:], dn_pv, preferred_element_type=jnp.float32
            )
            m_run = m_new
        out_ref[...] = acc / l_run

    out = pl.pallas_call(
        body,
        out_shape=jax.ShapeDtypeStruct((g, s_per, d), jnp.float32),
        in_specs=[
            pl.BlockSpec(memory_space=pltpu.VMEM),
            pl.BlockSpec(memory_space=pltpu.VMEM),
        ],
        out_specs=pl.BlockSpec(memory_space=pltpu.VMEM),
        scratch_shapes=[
            pltpu.VMEM((N_DEV, 2 * g, s_per, d), jnp.float32),
            pltpu.SemaphoreType.DMA((N_DEV - 1,)),
            pltpu.SemaphoreType.DMA((N_DEV - 1,)),
        ],
        compiler_params=pltpu.CompilerParams(collective_id=0),
    )(Qg, KVg)

    return out.reshape(b, h, s_per, d).transpose(0, 2, 1, 3)
